# Initial kernel scaffold; baseline (speedup 1.0000x reference)
import jax
import jax.numpy as jnp
from jax.experimental import pallas as pl


def kernel(x, edge_index, pair_index, enc_W, enc_b, Wl0, bl0, Wr0, Wl1, bl1, Wr1, Wl2, bl2, Wr2, p1_W, p1_b, p2_W, p2_b):
    raise NotImplementedError("write your pallas kernel here")


# speedup vs baseline: 5.1178x; 5.1178x over previous
import functools

import jax
import jax.numpy as jnp
from jax import lax
from jax.experimental import pallas as pl
from jax.experimental.pallas import tpu as pltpu
from jax.experimental.pallas import tpu_sc as plsc

N = 10000
D = 64
E = 320000
NC, NS, K = 2, 16, 128
NW = NC * NS
C = 80
EP = NW * C * K
NPAD = 10240
RPT = NPAD // NS
P = 8192
PC = (2 * P) // (NW * K)

_MESH = plsc.VectorSubcoreMesh(core_axis_name="c", subcore_axis_name="s",
                               num_cores=NC, num_subcores=NS)


_SEGSUM_SCRATCH = [
    pltpu.VMEM((C, K), jnp.int32),
    pltpu.VMEM((C, K), jnp.int32),
    pltpu.VMEM((K, D), jnp.float32),
    pltpu.VMEM((K, D), jnp.float32),
    pltpu.VMEM_SHARED((NPAD, D), jnp.float32),
    pltpu.SemaphoreType.DMA,
    pltpu.SemaphoreType.DMA,
]


def _sc_segsum_body(g_h, src_h, dst_h, zer_h, out_h,
                    src_v, dst_v, rows0, rows1, acc, sem0, sem1):
    cid = lax.axis_index("c")
    sid = lax.axis_index("s")
    wid = sid * NC + cid
    r0 = sid * RPT
    pltpu.sync_copy(zer_h.at[pl.ds(r0, RPT)], acc.at[pl.ds(r0, RPT)])
    pltpu.sync_copy(src_h.at[wid], src_v)
    pltpu.sync_copy(dst_h.at[wid], dst_v)
    plsc.subcore_barrier()

    pltpu.async_copy(g_h.at[src_v.at[0]], rows0, sem0)

    def body(j, carry):
        pltpu.async_copy(g_h.at[src_v.at[2 * j + 1]], rows1, sem1)
        pltpu.make_async_copy(g_h.at[src_v.at[2 * j]], rows0, sem0).wait()
        pltpu.sync_copy(rows0, acc.at[dst_v.at[2 * j]], add=True)

        @pl.when(j + 1 < C // 2)
        def _():
            pltpu.async_copy(g_h.at[src_v.at[2 * j + 2]], rows0, sem0)

        pltpu.make_async_copy(g_h.at[src_v.at[2 * j + 1]], rows1, sem1).wait()
        pltpu.sync_copy(rows1, acc.at[dst_v.at[2 * j + 1]], add=True)
        return carry

    lax.fori_loop(0, C // 2, body, 0)
    plsc.subcore_barrier()
    pltpu.sync_copy(acc.at[pl.ds(r0, RPT)],
                    out_h.at[pl.ds(cid * NPAD + r0, RPT)])


_SC_PARAMS = pltpu.CompilerParams(use_tc_tiling_on_sc=False)

_sc_segsum = pl.kernel(
    _sc_segsum_body,
    out_type=jax.ShapeDtypeStruct((2 * NPAD, D), jnp.float32),
    mesh=_MESH,
    scratch_types=_SEGSUM_SCRATCH,
    compiler_params=_SC_PARAMS,
)

_COUNT_SCRATCH = [
    pltpu.VMEM((C, K), jnp.int32),
    pltpu.VMEM((K,), jnp.float32),
    pltpu.VMEM_SHARED((NPAD,), jnp.float32),
]


def _sc_count_body(dst_h, zer_h, out_h, dst_v, ones_v, acc):
    cid = lax.axis_index("c")
    sid = lax.axis_index("s")
    wid = sid * NC + cid
    r0 = sid * RPT
    pltpu.sync_copy(zer_h.at[pl.ds(r0, RPT)], acc.at[pl.ds(r0, RPT)])
    for i in range(K // 16):
        ones_v[pl.ds(16 * i, 16)] = jnp.full((16,), 1.0, jnp.float32)
    pltpu.sync_copy(dst_h.at[wid], dst_v)
    plsc.subcore_barrier()

    def body(j, carry):
        pltpu.sync_copy(ones_v, acc.at[dst_v.at[j]], add=True)
        return carry

    lax.fori_loop(0, C, body, 0)
    plsc.subcore_barrier()
    pltpu.sync_copy(acc.at[pl.ds(r0, RPT)],
                    out_h.at[pl.ds(cid * NPAD + r0, RPT)])


_sc_count = pl.kernel(
    _sc_count_body,
    out_type=jax.ShapeDtypeStruct((2 * NPAD,), jnp.float32),
    mesh=_MESH,
    scratch_types=_COUNT_SCRATCH,
    compiler_params=_SC_PARAMS,
)

_PAIR_SCRATCH = [
    pltpu.VMEM((PC, K), jnp.int32),
    pltpu.VMEM((K, D), jnp.float32),
    pltpu.SemaphoreType.DMA,
]


def _sc_pair_gather_body(h_h, idx_h, out_h, idx_v, rows_v, sem):
    cid = lax.axis_index("c")
    sid = lax.axis_index("s")
    wid = sid * NC + cid
    pltpu.sync_copy(idx_h.at[wid], idx_v)
    for j in range(PC):
        pltpu.async_copy(h_h.at[idx_v.at[j]], rows_v, sem).wait()
        pltpu.sync_copy(rows_v, out_h.at[pl.ds(wid * (PC * K) + j * K, K)])


_sc_pair_gather = pl.kernel(
    _sc_pair_gather_body,
    out_type=jax.ShapeDtypeStruct((2 * P, D), jnp.float32),
    mesh=_MESH,
    scratch_types=_PAIR_SCRATCH,
    compiler_params=_SC_PARAMS,
)


def _tc_enc_body(x_ref, ewT, eb, wlT, wrT, bl, g_ref, r_ref):
    h = jnp.maximum(x_ref[...] @ ewT[...] + eb[...], 0.0)
    g_ref[...] = h @ wlT[...]
    r_ref[...] = h @ wrT[...] + bl[...]


_tc_enc = pl.pallas_call(
    _tc_enc_body,
    out_shape=(jax.ShapeDtypeStruct((N, D), jnp.float32),
               jax.ShapeDtypeStruct((N, D), jnp.float32)),
)


def _combine(aggp_ref, cnt_ref):
    agg = aggp_ref[0:N, :] + aggp_ref[NPAD:NPAD + N, :]
    c = cnt_ref[...]
    cnt = jnp.maximum(c[0, :N] + c[1, :N], 1.0)
    return agg * (1.0 / cnt)[:, None]


def _tc_mid_body(aggp_ref, cnt_ref, r_ref, wlT, wrT, bl, g_ref, rn_ref):
    h = jnp.maximum(_combine(aggp_ref, cnt_ref) + r_ref[...], 0.0)
    g_ref[...] = h @ wlT[...]
    rn_ref[...] = h @ wrT[...] + bl[...]


_tc_mid = pl.pallas_call(
    _tc_mid_body,
    out_shape=(jax.ShapeDtypeStruct((N, D), jnp.float32),
               jax.ShapeDtypeStruct((N, D), jnp.float32)),
)


def _tc_last_body(aggp_ref, cnt_ref, r_ref, h_ref):
    h_ref[...] = jnp.maximum(_combine(aggp_ref, cnt_ref) + r_ref[...], 0.0)


_tc_last = pl.pallas_call(
    _tc_last_body,
    out_shape=jax.ShapeDtypeStruct((N, D), jnp.float32),
)


def _tc_pred_body(huv_ref, p1aT, p1bT, p1b, p2r, p2b, out_ref):
    hu = huv_ref[0:P, :]
    hv = huv_ref[P:2 * P, :]
    t = jnp.maximum(hu @ p1aT[...] + hv @ p1bT[...] + p1b[...], 0.0)
    out_ref[...] = jnp.sum(t * p2r[...], axis=1, keepdims=True) + p2b[...]


_tc_pred = pl.pallas_call(
    _tc_pred_body,
    out_shape=jax.ShapeDtypeStruct((P, 1), jnp.float32),
)


def kernel(x, edge_index, pair_index, enc_W, enc_b, Wl0, bl0, Wr0,
           Wl1, bl1, Wr1, Wl2, bl2, Wr2, p1_W, p1_b, p2_W, p2_b):
    src, dst = edge_index[0], edge_index[1]
    pad = EP - E
    srcp = jnp.concatenate([src, jnp.zeros((pad,), jnp.int32)]).reshape(NW, C, K)
    dstp = jnp.concatenate([dst, jnp.full((pad,), N, jnp.int32)]).reshape(NW, C, K)
    zer2 = jnp.zeros((NPAD, D), jnp.float32)
    zer1 = jnp.zeros((NPAD,), jnp.float32)

    cntp = _sc_count(dstp, zer1).reshape(2, NPAD)
    g, r = _tc_enc(x, enc_W.T, enc_b.reshape(1, D),
                   Wl0.T, Wr0.T, bl0.reshape(1, D))
    for Wl, bl, Wr in ((Wl1, bl1, Wr1), (Wl2, bl2, Wr2)):
        aggp = _sc_segsum(g, srcp, dstp, zer2)
        g, r = _tc_mid(aggp, cntp, r, Wl.T, Wr.T, bl.reshape(1, D))
    aggp = _sc_segsum(g, srcp, dstp, zer2)
    h = _tc_last(aggp, cntp, r)

    uv = jnp.concatenate([pair_index[:, 0], pair_index[:, 1]]).reshape(NW, PC, K)
    huv = _sc_pair_gather(h, uv)
    out = _tc_pred(huv, p1_W[:, :D].T, p1_W[:, D:].T, p1_b.reshape(1, D),
                   p2_W.reshape(1, D), p2_b.reshape(1, 1))
    return out.reshape(P)
